# baseline (speedup 1.0000x reference)
import functools

import jax
import jax.numpy as jnp
from jax import lax
from jax.experimental import pallas as pl
from jax.experimental.pallas import tpu as pltpu
from jax.experimental.pallas import tpu_sc as plsc

N = 100000
D = 128
S = 256
NC = 2
NS = 16
NW = NC * NS
SPW = S // NW
TILE = 256
LANES = 16
NVEC = D // LANES
NEG_INF = float("-inf")
INT_MIN = -(2**31)

_mesh = plsc.VectorSubcoreMesh(core_axis_name="c", subcore_axis_name="s")


@functools.partial(
    pl.kernel,
    mesh=_mesh,
    out_type=jax.ShapeDtypeStruct((S, D), jnp.float32),
    scratch_types=[
        pltpu.VMEM((128,), jnp.int32),
        pltpu.VMEM((128,), jnp.int32),
        pltpu.VMEM((128,), jnp.int32),
        pltpu.VMEM((128,), jnp.int32),
        pltpu.VMEM((128,), jnp.int32),
        pltpu.VMEM((128,), jnp.int32),
        pltpu.VMEM((128,), jnp.int32),
        pltpu.VMEM((128,), jnp.int32),
        pltpu.VMEM((528,), jnp.int32),
        pltpu.VMEM((2320,), jnp.int32),
        pltpu.VMEM((TILE, D), jnp.float32),
        pltpu.VMEM((TILE, D), jnp.float32),
        pltpu.VMEM((TILE, D), jnp.float32),
        pltpu.VMEM((SPW, D), jnp.float32),
        pltpu.SemaphoreType.DMA,
        pltpu.SemaphoreType.DMA,
        pltpu.SemaphoreType.DMA,
        pltpu.SemaphoreType.DMA,
    ],
)
def _segmax(feats_hbm, batch_hbm, out_hbm, idxg0, idxg1, idxg2, idxg3,
            sg0, sg1, sg2, sg3, scat, wbuf, fbuf0, fbuf1,
            fbuf2, acc, sem, fsem0, fsem1, fsem2):
    wid = lax.axis_index("s") * NC + lax.axis_index("c")
    seg_lo = wid * SPW
    lane = lax.broadcasted_iota(jnp.int32, (LANES,), 0)

    G = 256
    MS = 512

    idxg = (idxg0, idxg1, idxg2, idxg3)
    sg = (sg0, sg1, sg2, sg3)
    for k in range(4):
        for v in range(8):
            i0 = k * 128 + v * LANES
            idxg[k][pl.ds(v * LANES, LANES)] = jnp.minimum((i0 + lane) * G, N - 1)
    for k in range(4):
        pltpu.async_copy(batch_hbm.at[idxg[k]], sg[k], sem)
    for k in range(4):
        pltpu.make_async_copy(batch_hbm.at[idxg[k]], sg[k], sem).wait()
    for k in range(4):
        for v in range(8):
            scat[pl.ds(k * 128 + v * LANES, LANES)] = sg[k][pl.ds(v * LANES, LANES)]

    def scalar_lower_bound(read, size, steps, t):
        lo = jnp.int32(0)
        hi = jnp.int32(size)
        for _ in range(steps):
            mid = lax.shift_right_logical(lo + hi, 1)
            v = read(jnp.minimum(mid, size - 1))
            ge = v >= t
            live = lo < hi
            lo = jnp.where(jnp.logical_and(live, jnp.logical_not(ge)), mid + 1, lo)
            hi = jnp.where(jnp.logical_and(live, ge), mid, hi)
        return lo

    b = []
    base_s = []
    for k in range(SPW + 1):
        w = scalar_lower_bound(
            lambda m: scat[pl.ds(m, LANES)][0], MS, 10, seg_lo + k
        )
        base_s.append(jnp.minimum((jnp.maximum(w, 1) - 1) * G, N - G))

    for k in range(SPW + 1):
        pltpu.async_copy(
            batch_hbm.at[pl.ds(pl.multiple_of(base_s[k], 8), G)],
            wbuf.at[pl.ds(k * G, G)], sem,
        )
    for k in range(SPW + 1):
        pltpu.make_async_copy(
            batch_hbm.at[pl.ds(pl.multiple_of(base_s[k], 8), G)],
            wbuf.at[pl.ds(k * G, G)], sem,
        ).wait()

    for k in range(SPW + 1):
        lb = scalar_lower_bound(
            lambda m, k=k: wbuf[pl.ds(k * G + m, LANES)][0], G, 9, seg_lo + k
        )
        b.append(base_s[k] + lb)

    neg = jnp.full((LANES,), NEG_INF, jnp.float32)
    for si in range(SPW):
        for j in range(NVEC):
            acc[si, pl.ds(j * LANES, LANES)] = neg

    b0a = (b[0] // 8) * 8
    ntiles = (b[SPW] - b0a + TILE - 1) // TILE
    bufs = (fbuf0, fbuf1, fbuf2)
    sems = (fsem0, fsem1, fsem2)
    nbuf = len(bufs)

    def tile_base(t):
        return pl.multiple_of(jnp.minimum(b0a + t * TILE, N - TILE), 8)

    def issue(t, buf, fsem):
        pltpu.async_copy(feats_hbm.at[pl.ds(tile_base(t), TILE)], buf, fsem)

    def wait(t, buf, fsem):
        pltpu.make_async_copy(
            feats_hbm.at[pl.ds(tile_base(t), TILE)], buf, fsem
        ).wait()

    def process(t, buf):
        base = tile_base(t)
        for si in range(SPW):
            r0 = jnp.maximum(b[si] - base, 0)
            r1 = jnp.minimum(b[si + 1] - base, TILE)

            @pl.when(r1 > r0)
            def _(si=si, r0=r0, r1=r1):
                carry = tuple(acc[si, pl.ds(j * LANES, LANES)] for j in range(NVEC))

                @plsc.parallel_loop(r0, r1, carry=carry, unroll=4)
                def res(r, c):
                    return tuple(
                        jnp.maximum(c[j], buf[r, pl.ds(j * LANES, LANES)])
                        for j in range(NVEC)
                    )
                for j in range(NVEC):
                    acc[si, pl.ds(j * LANES, LANES)] = res[j]

    for k in range(nbuf):

        @pl.when(k < ntiles)
        def _(k=k):
            issue(k, bufs[k], sems[k])

    def group_body(g, _):
        for k in range(nbuf):
            t = g * nbuf + k

            @pl.when(t < ntiles)
            def _(t=t, k=k):
                wait(t, bufs[k], sems[k])
                process(t, bufs[k])

                @pl.when(t + nbuf < ntiles)
                def _():
                    issue(t + nbuf, bufs[k], sems[k])

        return 0

    lax.fori_loop(0, (ntiles + nbuf - 1) // nbuf, group_body, 0)
    pltpu.sync_copy(acc, out_hbm.at[pl.ds(pl.multiple_of(seg_lo, 8), SPW)])


def kernel(feats, batch):
    return _segmax(feats, batch.astype(jnp.int32))


# speedup vs baseline: 1.0285x; 1.0285x over previous
import functools

import jax
import jax.numpy as jnp
from jax import lax
from jax.experimental import pallas as pl
from jax.experimental.pallas import tpu as pltpu
from jax.experimental.pallas import tpu_sc as plsc

N = 100000
D = 128
S = 256
NC = 2
NS = 16
NW = NC * NS
SPW = S // NW
TILE = 256
LANES = 16
NVEC = D // LANES
NEG_INF = float("-inf")
INT_MIN = -(2**31)

_mesh = plsc.VectorSubcoreMesh(core_axis_name="c", subcore_axis_name="s")


@functools.partial(
    pl.kernel,
    mesh=_mesh,
    out_type=jax.ShapeDtypeStruct((S, D), jnp.float32),
    scratch_types=[
        pltpu.VMEM((128,), jnp.int32),
        pltpu.VMEM((128,), jnp.int32),
        pltpu.VMEM((128,), jnp.int32),
        pltpu.VMEM((128,), jnp.int32),
        pltpu.VMEM((128,), jnp.int32),
        pltpu.VMEM((128,), jnp.int32),
        pltpu.VMEM((128,), jnp.int32),
        pltpu.VMEM((128,), jnp.int32),
        pltpu.VMEM((528,), jnp.int32),
        pltpu.VMEM((2320,), jnp.int32),
        pltpu.VMEM((TILE, D), jnp.float32),
        pltpu.VMEM((TILE, D), jnp.float32),
        pltpu.VMEM((TILE, D), jnp.float32),
        pltpu.VMEM((SPW, D), jnp.float32),
        pltpu.SemaphoreType.DMA,
        pltpu.SemaphoreType.DMA,
        pltpu.SemaphoreType.DMA,
        pltpu.SemaphoreType.DMA,
    ],
)
def _segmax(feats_hbm, batch_hbm, out_hbm, idxg0, idxg1, idxg2, idxg3,
            sg0, sg1, sg2, sg3, scat, wbuf, fbuf0, fbuf1,
            fbuf2, acc, sem, fsem0, fsem1, fsem2):
    wid = lax.axis_index("s") * NC + lax.axis_index("c")
    seg_lo = wid * SPW
    lane = lax.broadcasted_iota(jnp.int32, (LANES,), 0)

    G = 256
    MS = 512

    idxg = (idxg0, idxg1, idxg2, idxg3)
    sg = (sg0, sg1, sg2, sg3)
    for k in range(4):
        for v in range(8):
            i0 = k * 128 + v * LANES
            idxg[k][pl.ds(v * LANES, LANES)] = jnp.minimum((i0 + lane) * G, N - 1)
    for k in range(4):
        pltpu.async_copy(batch_hbm.at[idxg[k]], sg[k], sem)
    for k in range(4):
        pltpu.make_async_copy(batch_hbm.at[idxg[k]], sg[k], sem).wait()
    for k in range(4):
        for v in range(8):
            scat[pl.ds(k * 128 + v * LANES, LANES)] = sg[k][pl.ds(v * LANES, LANES)]

    def lower_bound_multi(read, size, steps, ts):
        nk = len(ts)
        lo = [jnp.int32(0)] * nk
        hi = [jnp.int32(size)] * nk
        for _ in range(steps):
            mid = [lax.shift_right_logical(lo[k] + hi[k], 1) for k in range(nk)]
            v = [read(jnp.minimum(mid[k], size - 1), k) for k in range(nk)]
            for k in range(nk):
                ge = v[k] >= ts[k]
                live = lo[k] < hi[k]
                lo[k] = jnp.where(
                    jnp.logical_and(live, jnp.logical_not(ge)), mid[k] + 1, lo[k]
                )
                hi[k] = jnp.where(jnp.logical_and(live, ge), mid[k], hi[k])
        return lo

    targets_s = [seg_lo + k for k in range(SPW + 1)]

    ws = lower_bound_multi(
        lambda m, k: scat[pl.ds(m, LANES)][0], MS, 10, targets_s
    )
    base_s = [
        jnp.minimum((jnp.maximum(w, 1) - 1) * G, N - G) for w in ws
    ]

    for k in range(SPW + 1):
        pltpu.async_copy(
            batch_hbm.at[pl.ds(pl.multiple_of(base_s[k], 8), G)],
            wbuf.at[pl.ds(k * G, G)], sem,
        )
    for k in range(SPW + 1):
        pltpu.make_async_copy(
            batch_hbm.at[pl.ds(pl.multiple_of(base_s[k], 8), G)],
            wbuf.at[pl.ds(k * G, G)], sem,
        ).wait()

    lbs = lower_bound_multi(
        lambda m, k: wbuf[pl.ds(k * G + m, LANES)][0], G, 9, targets_s
    )
    b = [base_s[k] + lbs[k] for k in range(SPW + 1)]

    neg = jnp.full((LANES,), NEG_INF, jnp.float32)
    for si in range(SPW):
        for j in range(NVEC):
            acc[si, pl.ds(j * LANES, LANES)] = neg

    b0a = (b[0] // 8) * 8
    ntiles = (b[SPW] - b0a + TILE - 1) // TILE
    bufs = (fbuf0, fbuf1, fbuf2)
    sems = (fsem0, fsem1, fsem2)
    nbuf = len(bufs)

    def tile_base(t):
        return pl.multiple_of(jnp.minimum(b0a + t * TILE, N - TILE), 8)

    def issue(t, buf, fsem):
        pltpu.async_copy(feats_hbm.at[pl.ds(tile_base(t), TILE)], buf, fsem)

    def wait(t, buf, fsem):
        pltpu.make_async_copy(
            feats_hbm.at[pl.ds(tile_base(t), TILE)], buf, fsem
        ).wait()

    def process(t, buf):
        base = tile_base(t)
        for si in range(SPW):
            r0 = jnp.maximum(b[si] - base, 0)
            r1 = jnp.minimum(b[si + 1] - base, TILE)

            @pl.when(r1 > r0)
            def _(si=si, r0=r0, r1=r1):
                carry = tuple(acc[si, pl.ds(j * LANES, LANES)] for j in range(NVEC))

                @plsc.parallel_loop(r0, r1, carry=carry, unroll=4)
                def res(r, c):
                    return tuple(
                        jnp.maximum(c[j], buf[r, pl.ds(j * LANES, LANES)])
                        for j in range(NVEC)
                    )
                for j in range(NVEC):
                    acc[si, pl.ds(j * LANES, LANES)] = res[j]

    for k in range(nbuf):

        @pl.when(k < ntiles)
        def _(k=k):
            issue(k, bufs[k], sems[k])

    def group_body(g, _):
        for k in range(nbuf):
            t = g * nbuf + k

            @pl.when(t < ntiles)
            def _(t=t, k=k):
                wait(t, bufs[k], sems[k])
                process(t, bufs[k])

                @pl.when(t + nbuf < ntiles)
                def _():
                    issue(t + nbuf, bufs[k], sems[k])

        return 0

    lax.fori_loop(0, (ntiles + nbuf - 1) // nbuf, group_body, 0)
    pltpu.sync_copy(acc, out_hbm.at[pl.ds(pl.multiple_of(seg_lo, 8), SPW)])


def kernel(feats, batch):
    return _segmax(feats, batch.astype(jnp.int32))


# speedup vs baseline: 1.4782x; 1.4373x over previous
import functools

import jax
import jax.numpy as jnp
from jax import lax
from jax.experimental import pallas as pl
from jax.experimental.pallas import tpu as pltpu
from jax.experimental.pallas import tpu_sc as plsc

N = 100000
D = 128
S = 256
NC = 2
NS = 16
NW = NC * NS
SPW = S // NW
TILE = 256
LANES = 16
NVEC = D // LANES
NEG_INF = float("-inf")
INT_MIN = -(2**31)

_mesh = plsc.VectorSubcoreMesh(core_axis_name="c", subcore_axis_name="s")


@functools.partial(
    pl.kernel,
    mesh=_mesh,
    out_type=jax.ShapeDtypeStruct((S, D), jnp.float32),
    scratch_types=[
        pltpu.VMEM((LANES,), jnp.int32),
        pltpu.VMEM((LANES,), jnp.int32),
        pltpu.VMEM_SHARED((N,), jnp.int32),
        pltpu.VMEM((6256,), jnp.int32),
        pltpu.VMEM((TILE, D), jnp.float32),
        pltpu.VMEM((TILE, D), jnp.float32),
        pltpu.VMEM((TILE, D), jnp.float32),
        pltpu.VMEM((SPW, D), jnp.float32),
        pltpu.SemaphoreType.DMA,
        pltpu.SemaphoreType.DMA,
        pltpu.SemaphoreType.DMA,
        pltpu.SemaphoreType.DMA,
    ],
)
def _segmax(feats_hbm, batch_hbm, out_hbm, idxref, probe, sbatch, bounce,
            fbuf0, fbuf1, fbuf2, acc, sem, fsem0, fsem1, fsem2):
    wid = lax.axis_index("s") * NC + lax.axis_index("c")
    seg_lo = wid * SPW
    lane = lax.broadcasted_iota(jnp.int32, (LANES,), 0)

    sid = lax.axis_index("s")
    CH = 6256
    @pl.when(sid < NS - 1)
    def _():
        cb = pl.multiple_of(sid * CH, 8)
        pltpu.sync_copy(batch_hbm.at[pl.ds(cb, CH)], bounce)
        pltpu.sync_copy(bounce, sbatch.at[pl.ds(cb, CH)])

    @pl.when(sid == NS - 1)
    def _():
        ct = N - (NS - 1) * CH
        pltpu.sync_copy(
            batch_hbm.at[pl.ds((NS - 1) * CH, ct)], bounce.at[pl.ds(0, ct)]
        )
        pltpu.sync_copy(
            bounce.at[pl.ds(0, ct)], sbatch.at[pl.ds((NS - 1) * CH, ct)]
        )

    plsc.subcore_barrier()

    targets = seg_lo + lane

    def sstep(_, lohi):
        lo, hi = lohi
        mid = lax.shift_right_logical(lo + hi, 1)
        mid_c = jnp.minimum(mid, N - 1)
        idxref[...] = mid_c
        pltpu.async_copy(sbatch.at[idxref], probe, sem).wait()
        ge = probe[...] >= targets
        live = lo < hi
        new_lo = jnp.where(ge, lo, mid + 1)
        new_hi = jnp.where(ge, mid, hi)
        return (jnp.where(live, new_lo, lo), jnp.where(live, new_hi, hi))

    bounds, _ = lax.fori_loop(
        0, 17, sstep,
        (jnp.zeros((LANES,), jnp.int32), jnp.full((LANES,), N, jnp.int32)),
    )
    b = [bounds[k] for k in range(SPW + 1)]

    neg = jnp.full((LANES,), NEG_INF, jnp.float32)
    for si in range(SPW):
        for j in range(NVEC):
            acc[si, pl.ds(j * LANES, LANES)] = neg

    b0a = (b[0] // 8) * 8
    ntiles = (b[SPW] - b0a + TILE - 1) // TILE
    bufs = (fbuf0, fbuf1, fbuf2)
    sems = (fsem0, fsem1, fsem2)
    nbuf = len(bufs)

    def tile_base(t):
        return pl.multiple_of(jnp.minimum(b0a + t * TILE, N - TILE), 8)

    def issue(t, buf, fsem):
        pltpu.async_copy(feats_hbm.at[pl.ds(tile_base(t), TILE)], buf, fsem)

    def wait(t, buf, fsem):
        pltpu.make_async_copy(
            feats_hbm.at[pl.ds(tile_base(t), TILE)], buf, fsem
        ).wait()

    def process(t, buf):
        base = tile_base(t)
        for si in range(SPW):
            r0 = jnp.maximum(b[si] - base, 0)
            r1 = jnp.minimum(b[si + 1] - base, TILE)

            @pl.when(r1 > r0)
            def _(si=si, r0=r0, r1=r1):
                carry = tuple(acc[si, pl.ds(j * LANES, LANES)] for j in range(NVEC))

                @plsc.parallel_loop(r0, r1, carry=carry, unroll=4)
                def res(r, c):
                    return tuple(
                        jnp.maximum(c[j], buf[r, pl.ds(j * LANES, LANES)])
                        for j in range(NVEC)
                    )
                for j in range(NVEC):
                    acc[si, pl.ds(j * LANES, LANES)] = res[j]

    for k in range(nbuf):

        @pl.when(k < ntiles)
        def _(k=k):
            issue(k, bufs[k], sems[k])

    def group_body(g, _):
        for k in range(nbuf):
            t = g * nbuf + k

            @pl.when(t < ntiles)
            def _(t=t, k=k):
                wait(t, bufs[k], sems[k])
                process(t, bufs[k])

                @pl.when(t + nbuf < ntiles)
                def _():
                    issue(t + nbuf, bufs[k], sems[k])

        return 0

    lax.fori_loop(0, (ntiles + nbuf - 1) // nbuf, group_body, 0)
    pltpu.sync_copy(acc, out_hbm.at[pl.ds(pl.multiple_of(seg_lo, 8), SPW)])


def kernel(feats, batch):
    return _segmax(feats, batch.astype(jnp.int32))
